# baseline (speedup 1.0000x reference)
import jax
import jax.numpy as jnp
from jax.experimental import pallas as pl
from jax.experimental.pallas import tpu as pltpu


def _tc_body(ts_ref, gam_ref, x_ref, n_ref, o_ref):
    b = pl.program_id(0)
    g = gam_ref[ts_ref[b]]
    o_ref[...] = jnp.sqrt(g) * x_ref[...] + jnp.sqrt(1.0 - g) * n_ref[...]


def kernel(x_start, timesteps, noise, gammas):
    B, C, H, W = x_start.shape
    ts = timesteps.reshape(B).astype(jnp.int32)

    grid_spec = pltpu.PrefetchScalarGridSpec(
        num_scalar_prefetch=2,
        grid=(B, C),
        in_specs=[
            pl.BlockSpec((1, 1, H, W), lambda b, c, t, g: (b, c, 0, 0)),
            pl.BlockSpec((1, 1, H, W), lambda b, c, t, g: (b, c, 0, 0)),
        ],
        out_specs=pl.BlockSpec((1, 1, H, W), lambda b, c, t, g: (b, c, 0, 0)),
    )
    return pl.pallas_call(
        _tc_body,
        grid_spec=grid_spec,
        out_shape=jax.ShapeDtypeStruct((B, C, H, W), jnp.float32),
    )(ts, gammas.astype(jnp.float32), x_start, noise)


# speedup vs baseline: 1.2617x; 1.2617x over previous
import jax
import jax.numpy as jnp
from jax.experimental import pallas as pl
from jax.experimental.pallas import tpu as pltpu


_BB = 2


def _tc_body(ts_ref, gam_ref, x_ref, n_ref, o_ref):
    i = pl.program_id(0)
    for j in range(_BB):
        g = gam_ref[ts_ref[i * _BB + j]]
        o_ref[j] = jnp.sqrt(g) * x_ref[j] + jnp.sqrt(1.0 - g) * n_ref[j]


def kernel(x_start, timesteps, noise, gammas):
    B, C, H, W = x_start.shape
    ts = timesteps.reshape(B).astype(jnp.int32)

    grid_spec = pltpu.PrefetchScalarGridSpec(
        num_scalar_prefetch=2,
        grid=(B // _BB,),
        in_specs=[
            pl.BlockSpec((_BB, C, H, W), lambda b, t, g: (b, 0, 0, 0)),
            pl.BlockSpec((_BB, C, H, W), lambda b, t, g: (b, 0, 0, 0)),
        ],
        out_specs=pl.BlockSpec((_BB, C, H, W), lambda b, t, g: (b, 0, 0, 0)),
    )
    return pl.pallas_call(
        _tc_body,
        grid_spec=grid_spec,
        out_shape=jax.ShapeDtypeStruct((B, C, H, W), jnp.float32),
    )(ts, gammas.astype(jnp.float32), x_start, noise)


# speedup vs baseline: 1.2676x; 1.0047x over previous
import jax
import jax.numpy as jnp
from jax import lax
from jax.experimental import pallas as pl
from jax.experimental.pallas import tpu as pltpu

_NBUF = 4


def _body(ts_ref, gam_ref, x_hbm, n_hbm, o_hbm, xb, nb, ob, xsem, nsem, osem):
    nchunks = x_hbm.shape[0]

    def start_in(i, slot):
        pltpu.make_async_copy(x_hbm.at[i], xb.at[slot], xsem.at[slot]).start()
        pltpu.make_async_copy(n_hbm.at[i], nb.at[slot], nsem.at[slot]).start()

    for i in range(_NBUF):
        start_in(i, i)

    def step(i, _):
        slot = lax.rem(i, _NBUF)
        pltpu.make_async_copy(x_hbm.at[i], xb.at[slot], xsem.at[slot]).wait()
        pltpu.make_async_copy(n_hbm.at[i], nb.at[slot], nsem.at[slot]).wait()

        @pl.when(i >= _NBUF)
        def _():
            pltpu.make_async_copy(
                ob.at[slot], o_hbm.at[i - _NBUF], osem.at[slot]
            ).wait()

        g = gam_ref[ts_ref[i]]
        ob[slot] = jnp.sqrt(g) * xb[slot] + jnp.sqrt(1.0 - g) * nb[slot]
        pltpu.make_async_copy(ob.at[slot], o_hbm.at[i], osem.at[slot]).start()

        @pl.when(i + _NBUF < nchunks)
        def _():
            start_in(i + _NBUF, slot)

        return 0

    lax.fori_loop(0, nchunks, step, 0)

    def drain(i, _):
        slot = lax.rem(i, _NBUF)
        pltpu.make_async_copy(ob.at[slot], o_hbm.at[i], osem.at[slot]).wait()
        return 0

    lax.fori_loop(nchunks - _NBUF, nchunks, drain, 0)


def kernel(x_start, timesteps, noise, gammas):
    B, C, H, W = x_start.shape
    ts = timesteps.reshape(B).astype(jnp.int32)

    return pl.pallas_call(
        _body,
        grid=(),
        in_specs=[
            pl.BlockSpec(memory_space=pltpu.SMEM),
            pl.BlockSpec(memory_space=pltpu.SMEM),
            pl.BlockSpec(memory_space=pltpu.HBM),
            pl.BlockSpec(memory_space=pltpu.HBM),
        ],
        out_specs=pl.BlockSpec(memory_space=pltpu.HBM),
        scratch_shapes=[
            pltpu.VMEM((_NBUF, C, H, W), jnp.float32),
            pltpu.VMEM((_NBUF, C, H, W), jnp.float32),
            pltpu.VMEM((_NBUF, C, H, W), jnp.float32),
            pltpu.SemaphoreType.DMA((_NBUF,)),
            pltpu.SemaphoreType.DMA((_NBUF,)),
            pltpu.SemaphoreType.DMA((_NBUF,)),
        ],
        out_shape=jax.ShapeDtypeStruct((B, C, H, W), jnp.float32),
    )(ts, gammas.astype(jnp.float32), x_start, noise)
